# baseline (speedup 1.0000x reference)
import functools

import jax
import jax.numpy as jnp
from jax import lax
from jax.experimental import pallas as pl
from jax.experimental.pallas import tpu as pltpu
from jax.experimental.pallas import tpu_sc as plsc

N = 100000
E = 3200000
D = 16

NC = 2
NS = 16
NW = NC * NS

N_PAD = 100352
ROWS_PER_TILE = N_PAD // NS

S = 80
K = 8
C = S * K
N_CHUNKS = E // C

_mesh = plsc.VectorSubcoreMesh(core_axis_name="c", subcore_axis_name="s")
_sc_params = pltpu.CompilerParams(
    use_tc_tiling_on_sc=False, needs_layout_passes=False)


def _worker_id():
    return lax.axis_index("s") * NC + lax.axis_index("c")


@functools.partial(
    pl.kernel,
    out_type=[
        jax.ShapeDtypeStruct((NC, N_PAD, D), jnp.float32),
        jax.ShapeDtypeStruct((NC, N_PAD), jnp.float32),
    ],
    mesh=_mesh,
    scratch_types=[
        pltpu.VMEM((K, S), jnp.int32),
        pltpu.VMEM((C,), jnp.float32),
        pltpu.VMEM((C, D), jnp.float32),
        pltpu.VMEM((C, D), jnp.float32),
        pltpu.VMEM_SHARED((N_PAD, D), jnp.float32),
        pltpu.VMEM_SHARED((N_PAD,), jnp.float32),
    ],
    compiler_params=_sc_params,
)
def _sc_scatter(col2d, mask_h, attr_h, out_sum, out_deg,
                col_v, mask_v, attr_v, val_v, acc_sum, acc_deg):
    cid = lax.axis_index("c")
    sid = lax.axis_index("s")
    wid = _worker_id()

    zero16 = jnp.zeros((D,), jnp.float32)

    def zrow(i, _):
        val_v[i, :] = zero16
        mask_v[pl.ds(i * 16, 16)] = jnp.zeros((16,), jnp.float32)
        return 0
    lax.fori_loop(0, C // 16, zrow, 0)

    def zrow2(i, _):
        val_v[i, :] = zero16
        return 0
    lax.fori_loop(C // 16, C, zrow2, 0)
    r0 = sid * ROWS_PER_TILE
    for j in range(ROWS_PER_TILE // C):
        pltpu.sync_copy(val_v.at[pl.ds(0, C), :],
                        acc_sum.at[pl.ds(r0 + j * C, C), :])
        pltpu.sync_copy(mask_v.at[pl.ds(0, C)],
                        acc_deg.at[pl.ds(r0 + j * C, C)])
    _rem = ROWS_PER_TILE - (ROWS_PER_TILE // C) * C
    _rbase = r0 + (ROWS_PER_TILE // C) * C
    pltpu.sync_copy(val_v.at[pl.ds(0, _rem), :],
                    acc_sum.at[pl.ds(_rbase, _rem), :])
    pltpu.sync_copy(mask_v.at[pl.ds(0, _rem)],
                    acc_deg.at[pl.ds(_rbase, _rem)])
    plsc.subcore_barrier()

    iota = lax.iota(jnp.int32, 16)
    cols = [jnp.full((16,), k, jnp.int32) for k in range(D)]
    niter = jnp.where(wid < N_CHUNKS % NW, N_CHUNKS // NW + 1, N_CHUNKS // NW)

    def chunk_body(i, _):
        chunk = wid + i * NW
        ebase = chunk * C
        crow = chunk * K
        pltpu.sync_copy(col2d.at[pl.ds(crow, K), :], col_v)
        pltpu.sync_copy(mask_h.at[pl.ds(ebase, C)], mask_v)
        pltpu.sync_copy(attr_h.at[pl.ds(ebase, C), :], attr_v)

        def group_body(g, _):
            m = mask_v[pl.ds(g * 16, 16)]
            rows = g * 16 + iota
            for k in range(D):
                a = plsc.load_gather(attr_v, [rows, cols[k]])
                plsc.store_scatter(val_v, [rows, cols[k]], a * m)
            return 0
        lax.fori_loop(0, C // 16, group_body, 0)

        for j in range(K):
            idx = col_v.at[j]
            pltpu.sync_copy(val_v.at[pl.ds(j * S, S), :],
                            acc_sum.at[idx], add=True)
            pltpu.sync_copy(mask_v.at[pl.ds(j * S, S)],
                            acc_deg.at[idx], add=True)
        return 0

    lax.fori_loop(0, niter, chunk_body, 0)
    plsc.subcore_barrier()

    pltpu.sync_copy(acc_sum.at[pl.ds(r0, ROWS_PER_TILE), :],
                    out_sum.at[cid, pl.ds(r0, ROWS_PER_TILE), :])
    pltpu.sync_copy(acc_deg.at[pl.ds(r0, ROWS_PER_TILE)],
                    out_deg.at[cid, pl.ds(r0, ROWS_PER_TILE)])


@functools.partial(
    pl.kernel,
    out_type=jax.ShapeDtypeStruct((D, E), jnp.float32),
    mesh=_mesh,
    scratch_types=[
        pltpu.VMEM((K, S), jnp.int32),
        pltpu.VMEM((K, S), jnp.int32),
        pltpu.VMEM((C, D), jnp.float32),
        pltpu.VMEM((D, C), jnp.float32),
    ],
    compiler_params=_sc_params,
)
def _sc_gather(row2d, col2d, p1_h, p2_h, out_h, row_v, col_v, gbuf, gt_v):
    wid = _worker_id()
    iota = lax.iota(jnp.int32, 16)
    cols = [jnp.full((16,), k, jnp.int32) for k in range(D)]
    niter = jnp.where(wid < N_CHUNKS % NW, N_CHUNKS // NW + 1, N_CHUNKS // NW)

    def chunk_body(i, _):
        chunk = wid + i * NW
        ebase = chunk * C
        crow = chunk * K
        pltpu.sync_copy(row2d.at[pl.ds(crow, K), :], row_v)
        pltpu.sync_copy(col2d.at[pl.ds(crow, K), :], col_v)
        for j in range(K):
            dst = gbuf.at[pl.ds(j * S, S), :]
            pltpu.sync_copy(p1_h.at[row_v.at[j]], dst)
            pltpu.sync_copy(p2_h.at[col_v.at[j]], dst, add=True)

        def group_body(g, _):
            rows = g * 16 + iota
            for k in range(D):
                v = plsc.load_gather(gbuf, [rows, cols[k]])
                gt_v[k, pl.ds(g * 16, 16)] = v
            return 0
        lax.fori_loop(0, C // 16, group_body, 0)

        for k in range(D):
            pltpu.sync_copy(gt_v.at[k], out_h.at[k, pl.ds(ebase, C)])
        return 0

    lax.fori_loop(0, niter, chunk_body, 0)


_BN = 5000


def _tc_node_body(p0, p1, d0, d1, x, w1t, w2t, nrep, o1, o2):
    s = p0[...] + p1[...]
    deg = d0[...] + d1[...]
    nr1 = s / (deg + 1.0)
    xb = x[...]
    nrep[:, :D] = nr1
    nrep[:, D:] = xb
    rep = jnp.concatenate([nr1, xb], axis=1)
    o1[...] = jnp.dot(rep, w1t[...], preferred_element_type=jnp.float32)
    o2[...] = jnp.dot(rep, w2t[...], preferred_element_type=jnp.float32)


def _tc_node(p0, p1, d0, d1, x, w1t, w2t):
    return pl.pallas_call(
        _tc_node_body,
        grid=(N // _BN,),
        in_specs=[
            pl.BlockSpec((_BN, D), lambda i: (i, 0)),
            pl.BlockSpec((_BN, D), lambda i: (i, 0)),
            pl.BlockSpec((_BN, 1), lambda i: (i, 0)),
            pl.BlockSpec((_BN, 1), lambda i: (i, 0)),
            pl.BlockSpec((_BN, D), lambda i: (i, 0)),
            pl.BlockSpec((2 * D, D), lambda i: (0, 0)),
            pl.BlockSpec((2 * D, D), lambda i: (0, 0)),
        ],
        out_specs=[
            pl.BlockSpec((_BN, 2 * D), lambda i: (i, 0)),
            pl.BlockSpec((_BN, D), lambda i: (i, 0)),
            pl.BlockSpec((_BN, D), lambda i: (i, 0)),
        ],
        out_shape=[
            jax.ShapeDtypeStruct((N, 2 * D), jnp.float32),
            jax.ShapeDtypeStruct((N, D), jnp.float32),
            jax.ShapeDtypeStruct((N, D), jnp.float32),
        ],
    )(p0, p1, d0, d1, x, w1t, w2t)


_BEW = 25600


def _tc_finish_body(gt, at, w3, b, out):
    out[...] = (gt[...] + b[...]
                + jnp.dot(w3[...], at[...],
                          preferred_element_type=jnp.float32))


def _tc_finish(gt, attrT, w3, b2):
    return pl.pallas_call(
        _tc_finish_body,
        grid=(E // _BEW,),
        in_specs=[
            pl.BlockSpec((D, _BEW), lambda i: (0, i)),
            pl.BlockSpec((D, _BEW), lambda i: (0, i)),
            pl.BlockSpec((D, D), lambda i: (0, 0)),
            pl.BlockSpec((D, 1), lambda i: (0, 0)),
        ],
        out_specs=pl.BlockSpec((D, _BEW), lambda i: (0, i)),
        out_shape=jax.ShapeDtypeStruct((D, E), jnp.float32),
    )(gt, attrT, w3, b2)


def kernel(x, num_nodes, edge_index, edge_attr, mask, W, b):
    row2d = edge_index[0].reshape(E // S, S)
    col2d = edge_index[1].reshape(E // S, S)

    w1t = W[:, : 2 * D].T
    w2t = W[:, 2 * D : 4 * D].T
    w3 = W[:, 4 * D :]

    part_sum, part_deg = _sc_scatter(col2d, mask, edge_attr)
    node_rep, p1, p2 = _tc_node(
        part_sum[0], part_sum[1],
        part_deg[0][:, None], part_deg[1][:, None],
        x, w1t, w2t)
    gt = _sc_gather(row2d, col2d, p1, p2)
    edge_rep_t = _tc_finish(gt, edge_attr.T, w3, b[:, None])
    return (node_rep, edge_rep_t.T)


# speedup vs baseline: 1.5622x; 1.5622x over previous
import functools

import jax
import jax.numpy as jnp
from jax import lax
from jax.experimental import pallas as pl
from jax.experimental.pallas import tpu as pltpu
from jax.experimental.pallas import tpu_sc as plsc

N = 100000
E = 3200000
D = 16

NC = 2
NS = 16
NW = NC * NS

N_PAD = 100352
ROWS_PER_TILE = N_PAD // NS

S = 80
K = 8
C = S * K
N_CHUNKS = E // C

_mesh = plsc.VectorSubcoreMesh(core_axis_name="c", subcore_axis_name="s")
_sc_params = pltpu.CompilerParams(
    use_tc_tiling_on_sc=False, needs_layout_passes=False)


def _worker_id():
    return lax.axis_index("s") * NC + lax.axis_index("c")


@functools.partial(
    pl.kernel,
    out_type=[
        jax.ShapeDtypeStruct((NC, N_PAD, D), jnp.float32),
        jax.ShapeDtypeStruct((NC, N_PAD), jnp.float32),
    ],
    mesh=_mesh,
    scratch_types=[
        pltpu.VMEM((K, S), jnp.int32),
        pltpu.VMEM((C,), jnp.float32),
        pltpu.VMEM((C, D), jnp.float32),
        pltpu.VMEM((C, D), jnp.float32),
        pltpu.VMEM_SHARED((N_PAD, D), jnp.float32),
        pltpu.VMEM_SHARED((N_PAD,), jnp.float32),
    ],
    compiler_params=_sc_params,
)
def _sc_scatter(col2d, mask_h, attr_h, out_sum, out_deg,
                col_v, mask_v, attr_v, val_v, acc_sum, acc_deg):
    cid = lax.axis_index("c")
    sid = lax.axis_index("s")
    wid = _worker_id()

    zero16 = jnp.zeros((D,), jnp.float32)

    def zrow(i, _):
        val_v[i, :] = zero16
        mask_v[pl.ds(i * 16, 16)] = jnp.zeros((16,), jnp.float32)
        return 0
    lax.fori_loop(0, C // 16, zrow, 0)

    def zrow2(i, _):
        val_v[i, :] = zero16
        return 0
    lax.fori_loop(C // 16, C, zrow2, 0)
    r0 = sid * ROWS_PER_TILE
    for j in range(ROWS_PER_TILE // C):
        pltpu.sync_copy(val_v.at[pl.ds(0, C), :],
                        acc_sum.at[pl.ds(r0 + j * C, C), :])
        pltpu.sync_copy(mask_v.at[pl.ds(0, C)],
                        acc_deg.at[pl.ds(r0 + j * C, C)])
    _rem = ROWS_PER_TILE - (ROWS_PER_TILE // C) * C
    _rbase = r0 + (ROWS_PER_TILE // C) * C
    pltpu.sync_copy(val_v.at[pl.ds(0, _rem), :],
                    acc_sum.at[pl.ds(_rbase, _rem), :])
    pltpu.sync_copy(mask_v.at[pl.ds(0, _rem)],
                    acc_deg.at[pl.ds(_rbase, _rem)])
    plsc.subcore_barrier()

    iota = lax.iota(jnp.int32, 16)
    cols = [jnp.full((16,), k, jnp.int32) for k in range(D)]
    niter = jnp.where(wid < N_CHUNKS % NW, N_CHUNKS // NW + 1, N_CHUNKS // NW)

    def chunk_body(i, _):
        chunk = wid + i * NW
        ebase = chunk * C
        crow = chunk * K
        pltpu.sync_copy(col2d.at[pl.ds(crow, K), :], col_v)
        pltpu.sync_copy(mask_h.at[pl.ds(ebase, C)], mask_v)
        pltpu.sync_copy(attr_h.at[pl.ds(ebase, C), :], attr_v)

        def group_body(g, _):
            m = mask_v[pl.ds(g * 16, 16)]
            rows = g * 16 + iota
            for k in range(D):
                a = plsc.load_gather(attr_v, [rows, cols[k]])
                plsc.store_scatter(val_v, [rows, cols[k]], a * m)
            return 0
        lax.fori_loop(0, C // 16, group_body, 0)

        for j in range(K):
            idx = col_v.at[j]
            pltpu.sync_copy(val_v.at[pl.ds(j * S, S), :],
                            acc_sum.at[idx], add=True)
            pltpu.sync_copy(mask_v.at[pl.ds(j * S, S)],
                            acc_deg.at[idx], add=True)
        return 0

    lax.fori_loop(0, niter, chunk_body, 0)
    plsc.subcore_barrier()

    pltpu.sync_copy(acc_sum.at[pl.ds(r0, ROWS_PER_TILE), :],
                    out_sum.at[cid, pl.ds(r0, ROWS_PER_TILE), :])
    pltpu.sync_copy(acc_deg.at[pl.ds(r0, ROWS_PER_TILE)],
                    out_deg.at[cid, pl.ds(r0, ROWS_PER_TILE)])


@functools.partial(
    pl.kernel,
    out_type=jax.ShapeDtypeStruct((E, D), jnp.float32),
    mesh=_mesh,
    scratch_types=[
        pltpu.VMEM((K, S), jnp.int32),
        pltpu.VMEM((K, S), jnp.int32),
        pltpu.VMEM((C, D), jnp.float32),
    ],
    compiler_params=_sc_params,
)
def _sc_gather(row2d, col2d, p1_h, p2_h, out_h, row_v, col_v, gbuf):
    wid = _worker_id()
    niter = jnp.where(wid < N_CHUNKS % NW, N_CHUNKS // NW + 1, N_CHUNKS // NW)

    def chunk_body(i, _):
        chunk = wid + i * NW
        ebase = chunk * C
        crow = chunk * K
        pltpu.sync_copy(row2d.at[pl.ds(crow, K), :], row_v)
        pltpu.sync_copy(col2d.at[pl.ds(crow, K), :], col_v)
        for j in range(K):
            dst = gbuf.at[pl.ds(j * S, S), :]
            pltpu.sync_copy(p1_h.at[row_v.at[j]], dst)
            pltpu.sync_copy(p2_h.at[col_v.at[j]], dst, add=True)
        pltpu.sync_copy(gbuf, out_h.at[pl.ds(ebase, C), :])
        return 0

    lax.fori_loop(0, niter, chunk_body, 0)


_BN = 5000


def _tc_node_body(p0, p1, d0, d1, x, w1t, w2t, nrep, o1, o2):
    s = p0[...] + p1[...]
    deg = d0[...] + d1[...]
    nr1 = s / (deg + 1.0)
    xb = x[...]
    nrep[:, :D] = nr1
    nrep[:, D:] = xb
    rep = jnp.concatenate([nr1, xb], axis=1)
    o1[...] = jnp.dot(rep, w1t[...], preferred_element_type=jnp.float32)
    o2[...] = jnp.dot(rep, w2t[...], preferred_element_type=jnp.float32)


def _tc_node(p0, p1, d0, d1, x, w1t, w2t):
    return pl.pallas_call(
        _tc_node_body,
        grid=(N // _BN,),
        in_specs=[
            pl.BlockSpec((_BN, D), lambda i: (i, 0)),
            pl.BlockSpec((_BN, D), lambda i: (i, 0)),
            pl.BlockSpec((_BN, 1), lambda i: (i, 0)),
            pl.BlockSpec((_BN, 1), lambda i: (i, 0)),
            pl.BlockSpec((_BN, D), lambda i: (i, 0)),
            pl.BlockSpec((2 * D, D), lambda i: (0, 0)),
            pl.BlockSpec((2 * D, D), lambda i: (0, 0)),
        ],
        out_specs=[
            pl.BlockSpec((_BN, 2 * D), lambda i: (i, 0)),
            pl.BlockSpec((_BN, D), lambda i: (i, 0)),
            pl.BlockSpec((_BN, D), lambda i: (i, 0)),
        ],
        out_shape=[
            jax.ShapeDtypeStruct((N, 2 * D), jnp.float32),
            jax.ShapeDtypeStruct((N, D), jnp.float32),
            jax.ShapeDtypeStruct((N, D), jnp.float32),
        ],
    )(p0, p1, d0, d1, x, w1t, w2t)


_BEW = 25600


def _tc_finish_body(gt, at, w3, b, out):
    out[...] = (gt[...] + b[...]
                + jnp.dot(w3[...], at[...],
                          preferred_element_type=jnp.float32))


def _tc_finish(gt, attrT, w3, b2):
    return pl.pallas_call(
        _tc_finish_body,
        grid=(E // _BEW,),
        in_specs=[
            pl.BlockSpec((D, _BEW), lambda i: (0, i)),
            pl.BlockSpec((D, _BEW), lambda i: (0, i)),
            pl.BlockSpec((D, D), lambda i: (0, 0)),
            pl.BlockSpec((D, 1), lambda i: (0, 0)),
        ],
        out_specs=pl.BlockSpec((D, _BEW), lambda i: (0, i)),
        out_shape=jax.ShapeDtypeStruct((D, E), jnp.float32),
    )(gt, attrT, w3, b2)


def kernel(x, num_nodes, edge_index, edge_attr, mask, W, b):
    row2d = edge_index[0].reshape(E // S, S)
    col2d = edge_index[1].reshape(E // S, S)

    w1t = W[:, : 2 * D].T
    w2t = W[:, 2 * D : 4 * D].T
    w3 = W[:, 4 * D :]

    part_sum, part_deg = _sc_scatter(col2d, mask, edge_attr)
    node_rep, p1, p2 = _tc_node(
        part_sum[0], part_sum[1],
        part_deg[0][:, None], part_deg[1][:, None],
        x, w1t, w2t)
    g = _sc_gather(row2d, col2d, p1, p2)
    edge_rep_t = _tc_finish(g.T, edge_attr.T, w3, b[:, None])
    return (node_rep, edge_rep_t.T)


# speedup vs baseline: 2.0043x; 1.2830x over previous
import functools

import jax
import jax.numpy as jnp
from jax import lax
from jax.experimental import pallas as pl
from jax.experimental.pallas import tpu as pltpu
from jax.experimental.pallas import tpu_sc as plsc

N = 100000
E = 3200000
D = 16

NC = 2
NS = 16
NW = NC * NS

N_PAD = 100352
ROWS_PER_TILE = N_PAD // NS

S = 80
K = 8
C = S * K
N_CHUNKS = E // C

_mesh = plsc.VectorSubcoreMesh(core_axis_name="c", subcore_axis_name="s")
_sc_params = pltpu.CompilerParams(
    use_tc_tiling_on_sc=False, needs_layout_passes=False)


def _worker_id():
    return lax.axis_index("s") * NC + lax.axis_index("c")


@functools.partial(
    pl.kernel,
    out_type=[
        jax.ShapeDtypeStruct((NC, N_PAD, D), jnp.float32),
        jax.ShapeDtypeStruct((NC, N_PAD), jnp.float32),
    ],
    mesh=_mesh,
    scratch_types=[
        pltpu.VMEM((K, S), jnp.int32),
        pltpu.VMEM((C,), jnp.float32),
        pltpu.VMEM((C, D), jnp.float32),
        pltpu.VMEM((C, D), jnp.float32),
        pltpu.VMEM_SHARED((N_PAD, D), jnp.float32),
        pltpu.VMEM_SHARED((N_PAD,), jnp.float32),
    ],
    compiler_params=_sc_params,
)
def _sc_scatter(col2d, mask_h, attr_h, out_sum, out_deg,
                col_v, mask_v, attr_v, val_v, acc_sum, acc_deg):
    cid = lax.axis_index("c")
    sid = lax.axis_index("s")
    wid = _worker_id()

    zero16 = jnp.zeros((D,), jnp.float32)

    def zrow(i, _):
        val_v[i, :] = zero16
        mask_v[pl.ds(i * 16, 16)] = jnp.zeros((16,), jnp.float32)
        return 0
    lax.fori_loop(0, C // 16, zrow, 0)

    def zrow2(i, _):
        val_v[i, :] = zero16
        return 0
    lax.fori_loop(C // 16, C, zrow2, 0)
    r0 = sid * ROWS_PER_TILE
    for j in range(ROWS_PER_TILE // C):
        pltpu.sync_copy(val_v.at[pl.ds(0, C), :],
                        acc_sum.at[pl.ds(r0 + j * C, C), :])
        pltpu.sync_copy(mask_v.at[pl.ds(0, C)],
                        acc_deg.at[pl.ds(r0 + j * C, C)])
    _rem = ROWS_PER_TILE - (ROWS_PER_TILE // C) * C
    _rbase = r0 + (ROWS_PER_TILE // C) * C
    pltpu.sync_copy(val_v.at[pl.ds(0, _rem), :],
                    acc_sum.at[pl.ds(_rbase, _rem), :])
    pltpu.sync_copy(mask_v.at[pl.ds(0, _rem)],
                    acc_deg.at[pl.ds(_rbase, _rem)])
    plsc.subcore_barrier()

    iota = lax.iota(jnp.int32, 16)
    cols = [jnp.full((16,), k, jnp.int32) for k in range(D)]
    niter = jnp.where(wid < N_CHUNKS % NW, N_CHUNKS // NW + 1, N_CHUNKS // NW)

    def chunk_body(i, _):
        chunk = wid + i * NW
        ebase = chunk * C
        crow = chunk * K
        pltpu.sync_copy(col2d.at[pl.ds(crow, K), :], col_v)
        pltpu.sync_copy(mask_h.at[pl.ds(ebase, C)], mask_v)
        pltpu.sync_copy(attr_h.at[pl.ds(ebase, C), :], attr_v)

        def group_body(g, _):
            m = mask_v[pl.ds(g * 16, 16)]
            rows = g * 16 + iota
            for k in range(D):
                a = plsc.load_gather(attr_v, [rows, cols[k]])
                plsc.store_scatter(val_v, [rows, cols[k]], a * m)
            return 0
        lax.fori_loop(0, C // 16, group_body, 0)

        for j in range(K):
            idx = col_v.at[j]
            pltpu.sync_copy(val_v.at[pl.ds(j * S, S), :],
                            acc_sum.at[idx], add=True)
            pltpu.sync_copy(mask_v.at[pl.ds(j * S, S)],
                            acc_deg.at[idx], add=True)
        return 0

    lax.fori_loop(0, niter, chunk_body, 0)
    plsc.subcore_barrier()

    pltpu.sync_copy(acc_sum.at[pl.ds(r0, ROWS_PER_TILE), :],
                    out_sum.at[cid, pl.ds(r0, ROWS_PER_TILE), :])
    pltpu.sync_copy(acc_deg.at[pl.ds(r0, ROWS_PER_TILE)],
                    out_deg.at[cid, pl.ds(r0, ROWS_PER_TILE)])


NBUF = 4
CNT_HI, CNT_LO = 160, 156


@functools.partial(
    pl.kernel,
    out_type=jax.ShapeDtypeStruct((E, D), jnp.float32),
    mesh=_mesh,
    scratch_types=[
        pltpu.VMEM((NBUF, K, S), jnp.int32),
        pltpu.VMEM((NBUF, K, S), jnp.int32),
        pltpu.VMEM((NBUF, C, D), jnp.float32),
    ] + [pltpu.SemaphoreType.DMA] * (2 * NBUF + 1),
    compiler_params=_sc_params,
)
def _sc_gather(row2d, col2d, p1_h, p2_h, out_h, row_v, col_v, gbuf,
               ix0, ix1, ix2, ix3, ot0, ot1, ot2, ot3, gsem):
    wid = _worker_id()
    ix = (ix0, ix1, ix2, ix3)
    ot = (ot0, ot1, ot2, ot3)
    start = jnp.where(wid < 2, wid * CNT_HI,
                      2 * CNT_HI + (wid - 2) * CNT_LO)
    cnt = jnp.where(wid < 2, CNT_HI, CNT_LO)

    def fire_idx(chunk, b):
        crow = chunk * K
        pltpu.async_copy(row2d.at[pl.ds(crow, K), :], row_v.at[b], ix[b])
        pltpu.async_copy(col2d.at[pl.ds(crow, K), :], col_v.at[b], ix[b])

    def wait_idx(b):
        pltpu.make_async_copy(row2d.at[pl.ds(0, K), :], row_v.at[b],
                              ix[b]).wait()
        pltpu.make_async_copy(col2d.at[pl.ds(0, K), :], col_v.at[b],
                              ix[b]).wait()

    def wait_out(b):
        pltpu.make_async_copy(gbuf.at[b], out_h.at[pl.ds(0, C), :],
                              ot[b]).wait()

    fire_idx(start, 0)
    fire_idx(start + 1, 1)

    def quad_body(i, _):
        for u in range(NBUF):
            b = u
            b2 = (u + 2) % NBUF
            c = i * NBUF + u
            chunk = start + c
            wait_idx(b)

            @pl.when(c >= NBUF)
            def _():
                wait_out(b)

            descs = []
            for j in range(K):
                pltpu.async_copy(p1_h.at[row_v.at[b, j]],
                                 gbuf.at[b, pl.ds(j * S, S), :], gsem)
            for j in range(K):
                pltpu.make_async_copy(p1_h.at[row_v.at[b, j]],
                                      gbuf.at[b, pl.ds(j * S, S), :],
                                      gsem).wait()
            for j in range(K):
                pltpu.async_copy(p2_h.at[col_v.at[b, j]],
                                 gbuf.at[b, pl.ds(j * S, S), :], gsem,
                                 add=True)
            for j in range(K):
                pltpu.make_async_copy(p2_h.at[col_v.at[b, j]],
                                      gbuf.at[b, pl.ds(j * S, S), :],
                                      gsem).wait()

            @pl.when(c + 2 < cnt)
            def _():
                fire_idx(chunk + 2, b2)

            pltpu.async_copy(gbuf.at[b], out_h.at[pl.ds(chunk * C, C), :],
                             ot[b])
        return 0

    lax.fori_loop(0, cnt // NBUF, quad_body, 0)
    for b in range(NBUF):
        wait_out(b)


_BN = 5000


def _tc_node_body(p0, p1, d0, d1, x, w1t, w2t, nrep, o1, o2):
    s = p0[...] + p1[...]
    deg = d0[...] + d1[...]
    nr1 = s / (deg + 1.0)
    xb = x[...]
    nrep[:, :D] = nr1
    nrep[:, D:] = xb
    rep = jnp.concatenate([nr1, xb], axis=1)
    o1[...] = jnp.dot(rep, w1t[...], preferred_element_type=jnp.float32)
    o2[...] = jnp.dot(rep, w2t[...], preferred_element_type=jnp.float32)


def _tc_node(p0, p1, d0, d1, x, w1t, w2t):
    return pl.pallas_call(
        _tc_node_body,
        grid=(N // _BN,),
        in_specs=[
            pl.BlockSpec((_BN, D), lambda i: (i, 0)),
            pl.BlockSpec((_BN, D), lambda i: (i, 0)),
            pl.BlockSpec((_BN, 1), lambda i: (i, 0)),
            pl.BlockSpec((_BN, 1), lambda i: (i, 0)),
            pl.BlockSpec((_BN, D), lambda i: (i, 0)),
            pl.BlockSpec((2 * D, D), lambda i: (0, 0)),
            pl.BlockSpec((2 * D, D), lambda i: (0, 0)),
        ],
        out_specs=[
            pl.BlockSpec((_BN, 2 * D), lambda i: (i, 0)),
            pl.BlockSpec((_BN, D), lambda i: (i, 0)),
            pl.BlockSpec((_BN, D), lambda i: (i, 0)),
        ],
        out_shape=[
            jax.ShapeDtypeStruct((N, 2 * D), jnp.float32),
            jax.ShapeDtypeStruct((N, D), jnp.float32),
            jax.ShapeDtypeStruct((N, D), jnp.float32),
        ],
    )(p0, p1, d0, d1, x, w1t, w2t)


_BEW = 25600


def _tc_finish_body(gt, at, w3, b, out):
    out[...] = (gt[...] + b[...]
                + jnp.dot(w3[...], at[...],
                          preferred_element_type=jnp.float32))


def _tc_finish(gt, attrT, w3, b2):
    return pl.pallas_call(
        _tc_finish_body,
        grid=(E // _BEW,),
        in_specs=[
            pl.BlockSpec((D, _BEW), lambda i: (0, i)),
            pl.BlockSpec((D, _BEW), lambda i: (0, i)),
            pl.BlockSpec((D, D), lambda i: (0, 0)),
            pl.BlockSpec((D, 1), lambda i: (0, 0)),
        ],
        out_specs=pl.BlockSpec((D, _BEW), lambda i: (0, i)),
        out_shape=jax.ShapeDtypeStruct((D, E), jnp.float32),
    )(gt, attrT, w3, b2)


def kernel(x, num_nodes, edge_index, edge_attr, mask, W, b):
    row2d = edge_index[0].reshape(E // S, S)
    col2d = edge_index[1].reshape(E // S, S)

    w1t = W[:, : 2 * D].T
    w2t = W[:, 2 * D : 4 * D].T
    w3 = W[:, 4 * D :]

    part_sum, part_deg = _sc_scatter(col2d, mask, edge_attr)
    node_rep, p1, p2 = _tc_node(
        part_sum[0], part_sum[1],
        part_deg[0][:, None], part_deg[1][:, None],
        x, w1t, w2t)
    g = _sc_gather(row2d, col2d, p1, p2)
    edge_rep_t = _tc_finish(g.T, edge_attr.T, w3, b[:, None])
    return (node_rep, edge_rep_t.T)


# speedup vs baseline: 2.1757x; 1.0855x over previous
import functools

import jax
import jax.numpy as jnp
from jax import lax
from jax.experimental import pallas as pl
from jax.experimental.pallas import tpu as pltpu
from jax.experimental.pallas import tpu_sc as plsc

N = 100000
E = 3200000
D = 16

NC = 2
NS = 16
NW = NC * NS

N_PAD = 100096
ROWS_PER_TILE = N_PAD // NS
D17 = D + 1

S = 80
K = 8
C = S * K
N_CHUNKS = E // C

_mesh = plsc.VectorSubcoreMesh(core_axis_name="c", subcore_axis_name="s")
_sc_params = pltpu.CompilerParams(
    use_tc_tiling_on_sc=False, needs_layout_passes=False)


def _worker_id():
    return lax.axis_index("s") * NC + lax.axis_index("c")


SC_HI, SC_LO = 158, 156


@functools.partial(
    pl.kernel,
    out_type=[
        jax.ShapeDtypeStruct((NC, N_PAD, D), jnp.float32),
        jax.ShapeDtypeStruct((NC, N_PAD), jnp.float32),
    ],
    mesh=_mesh,
    scratch_types=[
        pltpu.VMEM((2, K, S), jnp.int32),
        pltpu.VMEM((2, C), jnp.float32),
        pltpu.VMEM((2, C, D), jnp.float32),
        pltpu.VMEM_SHARED((N_PAD, D), jnp.float32),
        pltpu.VMEM_SHARED((N_PAD,), jnp.float32),
        pltpu.SemaphoreType.DMA,
        pltpu.SemaphoreType.DMA,
        pltpu.SemaphoreType.DMA,
        pltpu.SemaphoreType.DMA,
    ],
    compiler_params=_sc_params,
)
def _sc_scatter(col2d, mask_h, attr_h, out_sum, out_deg,
                col_v, mask_v, val_v, acc, acc_deg, in0, in1, sc0, sc1):
    cid = lax.axis_index("c")
    sid = lax.axis_index("s")
    wid = _worker_id()
    insem = (in0, in1)
    scsem = (sc0, sc1)

    zero16 = jnp.zeros((D,), jnp.float32)
    iota = lax.iota(jnp.int32, 16)
    cols = [jnp.full((16,), k, jnp.int32) for k in range(D)]

    def zrow(i, _):
        val_v[0, i, :] = zero16
        mask_v[0, pl.ds(i * 16, 16)] = jnp.zeros((16,), jnp.float32)
        return 0
    lax.fori_loop(0, C // 16, zrow, 0)

    def zrow2(i, _):
        val_v[0, i, :] = zero16
        return 0
    lax.fori_loop(C // 16, C, zrow2, 0)
    r0 = sid * ROWS_PER_TILE
    for j in range(ROWS_PER_TILE // C):
        pltpu.sync_copy(val_v.at[0, pl.ds(0, C), :],
                        acc.at[pl.ds(r0 + j * C, C), :])
        pltpu.sync_copy(mask_v.at[0, pl.ds(0, C)],
                        acc_deg.at[pl.ds(r0 + j * C, C)])
    _rem = ROWS_PER_TILE - (ROWS_PER_TILE // C) * C
    _rbase = r0 + (ROWS_PER_TILE // C) * C
    pltpu.sync_copy(val_v.at[0, pl.ds(0, _rem), :],
                    acc.at[pl.ds(_rbase, _rem), :])
    pltpu.sync_copy(mask_v.at[0, pl.ds(0, _rem)],
                    acc_deg.at[pl.ds(_rbase, _rem)])
    plsc.subcore_barrier()

    start = jnp.where(wid < 4, wid * SC_HI, 4 * SC_HI + (wid - 4) * SC_LO)
    cnt = jnp.where(wid < 4, SC_HI, SC_LO)

    def fire_in(chunk, b):
        ebase = chunk * C
        pltpu.async_copy(col2d.at[pl.ds(chunk * K, K), :], col_v.at[b],
                         insem[b])
        pltpu.async_copy(mask_h.at[pl.ds(ebase, C)], mask_v.at[b], insem[b])
        pltpu.async_copy(attr_h.at[pl.ds(ebase, C), :], val_v.at[b],
                         insem[b])

    def wait_in(b):
        pltpu.make_async_copy(col2d.at[pl.ds(0, K), :], col_v.at[b],
                              insem[b]).wait()
        pltpu.make_async_copy(mask_h.at[pl.ds(0, C)], mask_v.at[b],
                              insem[b]).wait()
        pltpu.make_async_copy(attr_h.at[pl.ds(0, C), :], val_v.at[b],
                              insem[b]).wait()

    def drain_sc(b):
        for j in range(K):
            pltpu.make_async_copy(val_v.at[b, pl.ds(j * S, S), :],
                                  acc.at[col_v.at[b, j]], scsem[b]).wait()
            pltpu.make_async_copy(mask_v.at[b, pl.ds(j * S, S)],
                                  acc_deg.at[col_v.at[b, j]], scsem[b]).wait()

    fire_in(start, 0)

    def pair_body(i, _):
        for b in range(2):
            c = i * 2 + b
            chunk = start + c
            wait_in(b)

            def group_body(g, _):
                m = mask_v[b, pl.ds(g * 16, 16)]
                rows = g * 16 + iota
                for k in range(D):
                    a = plsc.load_gather(val_v.at[b], [rows, cols[k]])
                    plsc.store_scatter(val_v.at[b], [rows, cols[k]], a * m)
                return 0
            lax.fori_loop(0, C // 16, group_body, 0)

            @pl.when(c >= 1)
            def _():
                drain_sc(1 - b)

            @pl.when(c + 1 < cnt)
            def _():
                fire_in(chunk + 1, 1 - b)

            for j in range(K):
                pltpu.async_copy(val_v.at[b, pl.ds(j * S, S), :],
                                 acc.at[col_v.at[b, j]], scsem[b], add=True)
                pltpu.async_copy(mask_v.at[b, pl.ds(j * S, S)],
                                 acc_deg.at[col_v.at[b, j]], scsem[b],
                                 add=True)
        return 0

    lax.fori_loop(0, cnt // 2, pair_body, 0)
    drain_sc(1)
    plsc.subcore_barrier()

    pltpu.sync_copy(acc.at[pl.ds(r0, ROWS_PER_TILE), :],
                    out_sum.at[cid, pl.ds(r0, ROWS_PER_TILE), :])
    pltpu.sync_copy(acc_deg.at[pl.ds(r0, ROWS_PER_TILE)],
                    out_deg.at[cid, pl.ds(r0, ROWS_PER_TILE)])


NBUF = 4
CNT_HI, CNT_LO = 160, 156


@functools.partial(
    pl.kernel,
    out_type=jax.ShapeDtypeStruct((E, D), jnp.float32),
    mesh=_mesh,
    scratch_types=[
        pltpu.VMEM((NBUF, K, S), jnp.int32),
        pltpu.VMEM((NBUF, K, S), jnp.int32),
        pltpu.VMEM((NBUF, C, D), jnp.float32),
    ] + [pltpu.SemaphoreType.DMA] * (2 * NBUF + 1),
    compiler_params=_sc_params,
)
def _sc_gather(row2d, col2d, p1_h, p2_h, out_h, row_v, col_v, gbuf,
               ix0, ix1, ix2, ix3, ot0, ot1, ot2, ot3, gsem):
    wid = _worker_id()
    ix = (ix0, ix1, ix2, ix3)
    ot = (ot0, ot1, ot2, ot3)
    start = jnp.where(wid < 2, wid * CNT_HI,
                      2 * CNT_HI + (wid - 2) * CNT_LO)
    cnt = jnp.where(wid < 2, CNT_HI, CNT_LO)

    def fire_idx(chunk, b):
        crow = chunk * K
        pltpu.async_copy(row2d.at[pl.ds(crow, K), :], row_v.at[b], ix[b])
        pltpu.async_copy(col2d.at[pl.ds(crow, K), :], col_v.at[b], ix[b])

    def wait_idx(b):
        pltpu.make_async_copy(row2d.at[pl.ds(0, K), :], row_v.at[b],
                              ix[b]).wait()
        pltpu.make_async_copy(col2d.at[pl.ds(0, K), :], col_v.at[b],
                              ix[b]).wait()

    def wait_out(b):
        pltpu.make_async_copy(gbuf.at[b], out_h.at[pl.ds(0, C), :],
                              ot[b]).wait()

    fire_idx(start, 0)
    fire_idx(start + 1, 1)

    def quad_body(i, _):
        for u in range(NBUF):
            b = u
            b2 = (u + 2) % NBUF
            c = i * NBUF + u
            chunk = start + c
            wait_idx(b)

            @pl.when(c >= NBUF)
            def _():
                wait_out(b)

            descs = []
            for j in range(K):
                pltpu.async_copy(p1_h.at[row_v.at[b, j]],
                                 gbuf.at[b, pl.ds(j * S, S), :], gsem)
            for j in range(K):
                pltpu.make_async_copy(p1_h.at[row_v.at[b, j]],
                                      gbuf.at[b, pl.ds(j * S, S), :],
                                      gsem).wait()
            for j in range(K):
                pltpu.async_copy(p2_h.at[col_v.at[b, j]],
                                 gbuf.at[b, pl.ds(j * S, S), :], gsem,
                                 add=True)
            for j in range(K):
                pltpu.make_async_copy(p2_h.at[col_v.at[b, j]],
                                      gbuf.at[b, pl.ds(j * S, S), :],
                                      gsem).wait()

            @pl.when(c + 2 < cnt)
            def _():
                fire_idx(chunk + 2, b2)

            pltpu.async_copy(gbuf.at[b], out_h.at[pl.ds(chunk * C, C), :],
                             ot[b])
        return 0

    lax.fori_loop(0, cnt // NBUF, quad_body, 0)
    for b in range(NBUF):
        wait_out(b)


_BN = 5000


def _tc_node_body(p0, p1, d0, d1, x, w1t, w2t, nrep, o1, o2):
    s = p0[...] + p1[...]
    deg = d0[...] + d1[...]
    nr1 = s / (deg + 1.0)
    xb = x[...]
    nrep[:, :D] = nr1
    nrep[:, D:] = xb
    rep = jnp.concatenate([nr1, xb], axis=1)
    o1[...] = jnp.dot(rep, w1t[...], preferred_element_type=jnp.float32)
    o2[...] = jnp.dot(rep, w2t[...], preferred_element_type=jnp.float32)


def _tc_node(p0, p1, d0, d1, x, w1t, w2t):
    return pl.pallas_call(
        _tc_node_body,
        grid=(N // _BN,),
        in_specs=[
            pl.BlockSpec((_BN, D), lambda i: (i, 0)),
            pl.BlockSpec((_BN, D), lambda i: (i, 0)),
            pl.BlockSpec((_BN, 1), lambda i: (i, 0)),
            pl.BlockSpec((_BN, 1), lambda i: (i, 0)),
            pl.BlockSpec((_BN, D), lambda i: (i, 0)),
            pl.BlockSpec((2 * D, D), lambda i: (0, 0)),
            pl.BlockSpec((2 * D, D), lambda i: (0, 0)),
        ],
        out_specs=[
            pl.BlockSpec((_BN, 2 * D), lambda i: (i, 0)),
            pl.BlockSpec((_BN, D), lambda i: (i, 0)),
            pl.BlockSpec((_BN, D), lambda i: (i, 0)),
        ],
        out_shape=[
            jax.ShapeDtypeStruct((N, 2 * D), jnp.float32),
            jax.ShapeDtypeStruct((N, D), jnp.float32),
            jax.ShapeDtypeStruct((N, D), jnp.float32),
        ],
    )(p0, p1, d0, d1, x, w1t, w2t)


_BEW = 25600


def _tc_finish_body(gt, at, w3, b, out):
    out[...] = (gt[...] + b[...]
                + jnp.dot(w3[...], at[...],
                          preferred_element_type=jnp.float32))


def _tc_finish(gt, attrT, w3, b2):
    return pl.pallas_call(
        _tc_finish_body,
        grid=(E // _BEW,),
        in_specs=[
            pl.BlockSpec((D, _BEW), lambda i: (0, i)),
            pl.BlockSpec((D, _BEW), lambda i: (0, i)),
            pl.BlockSpec((D, D), lambda i: (0, 0)),
            pl.BlockSpec((D, 1), lambda i: (0, 0)),
        ],
        out_specs=pl.BlockSpec((D, _BEW), lambda i: (0, i)),
        out_shape=jax.ShapeDtypeStruct((D, E), jnp.float32),
    )(gt, attrT, w3, b2)


def kernel(x, num_nodes, edge_index, edge_attr, mask, W, b):
    row2d = edge_index[0].reshape(E // S, S)
    col2d = edge_index[1].reshape(E // S, S)

    w1t = W[:, : 2 * D].T
    w2t = W[:, 2 * D : 4 * D].T
    w3 = W[:, 4 * D :]

    part_sum, part_deg = _sc_scatter(col2d, mask, edge_attr)
    node_rep, p1, p2 = _tc_node(
        part_sum[0], part_sum[1],
        part_deg[0][:, None], part_deg[1][:, None],
        x, w1t, w2t)
    g = _sc_gather(row2d, col2d, p1, p2)
    edge_rep_t = _tc_finish(g.T, edge_attr.T, w3, b[:, None])
    return (node_rep, edge_rep_t.T)
